# Initial kernel scaffold; baseline (speedup 1.0000x reference)
import jax
import jax.numpy as jnp
from jax.experimental import pallas as pl


def kernel(x, dr, field, jockey, horse, trainer, emb_dr_w, emb_field_w, emb_jockey_w, emb_horse_w, emb_trainer_w, W, b):
    raise NotImplementedError("write your pallas kernel here")


# speedup vs baseline: 1.0994x; 1.0994x over previous
import functools

import jax
import jax.numpy as jnp
from jax import lax
from jax.experimental import pallas as pl
from jax.experimental.pallas import tpu as pltpu
from jax.experimental.pallas import tpu_sc as plsc

B = 16384
N_NUM_FEATS = 64
K_FIELD = 16
K_ID = 32
OUT_DIM = N_NUM_FEATS + 2 * K_FIELD + 3 * K_ID

_info = plsc.get_sparse_core_info()
NC, NS, L = _info.num_cores, _info.num_subcores, _info.num_lanes
NW = NC * NS
BPW = B // NW


def _sc_kernel(x_h, dr_h, field_h, jockey_h, horse_h, trainer_h,
               ed_h, ef_h, ej_h, eh_h, et_h, w_h, b_h, out_h,
               x_v, dri_v, fi_v, ji_v, hi_v, ti_v,
               dr_v, fr_v, jr_v, hr_v, tr_v,
               w_v, b_v, out_v, sem):
    wid = lax.axis_index("s") * NC + lax.axis_index("c")
    base = wid * BPW

    pltpu.sync_copy(dr_h.at[pl.ds(base, BPW)], dri_v)
    pltpu.sync_copy(field_h.at[pl.ds(base, BPW)], fi_v)
    pltpu.sync_copy(jockey_h.at[pl.ds(base, BPW)], ji_v)
    pltpu.sync_copy(horse_h.at[pl.ds(base, BPW)], hi_v)
    pltpu.sync_copy(trainer_h.at[pl.ds(base, BPW)], ti_v)
    pltpu.sync_copy(w_h, w_v)
    pltpu.sync_copy(b_h, b_v)

    cps = [
        pltpu.async_copy(ed_h.at[dri_v], dr_v, sem),
        pltpu.async_copy(ef_h.at[fi_v], fr_v, sem),
        pltpu.async_copy(ej_h.at[ji_v], jr_v, sem),
        pltpu.async_copy(eh_h.at[hi_v], hr_v, sem),
        pltpu.async_copy(et_h.at[ti_v], tr_v, sem),
        pltpu.async_copy(x_h.at[pl.ds(base, BPW)], x_v, sem),
    ]
    for cp in cps:
        cp.wait()

    wc = [w_v[pl.ds(c * L, L)] for c in range(OUT_DIM // L)]

    bias = b_v[...]
    lane_iota = lax.iota(jnp.int32, L)
    perms = [lane_iota ^ s for s in (1, 2, 4, 8)]

    dnums = lax.GatherDimensionNumbers(
        offset_dims=(), collapsed_slice_dims=(0,), start_index_map=(0,))

    def _lane_sum(t):
        for p in perms:
            t = t + lax.gather(t, p[:, None], dnums, slice_sizes=(1,),
                               mode=lax.GatherScatterMode.PROMISE_IN_BOUNDS)
        return t

    def group_body(g, carry):
        row0 = g * L
        acc = jnp.zeros((L,), jnp.float32)
        for rl in range(L):
            r = row0 + rl
            t = jnp.maximum(x_v[r, pl.ds(0, L)], 0.0) * wc[0]
            t += jnp.maximum(x_v[r, pl.ds(L, L)], 0.0) * wc[1]
            t += jnp.maximum(x_v[r, pl.ds(2 * L, L)], 0.0) * wc[2]
            t += jnp.maximum(x_v[r, pl.ds(3 * L, L)], 0.0) * wc[3]
            t += jnp.maximum(dr_v[r, pl.ds(0, L)], 0.0) * wc[4]
            t += jnp.maximum(fr_v[r, pl.ds(0, L)], 0.0) * wc[5]
            t += jnp.maximum(jr_v[r, pl.ds(0, L)], 0.0) * wc[6]
            t += jnp.maximum(jr_v[r, pl.ds(L, L)], 0.0) * wc[7]
            t += jnp.maximum(hr_v[r, pl.ds(0, L)], 0.0) * wc[8]
            t += jnp.maximum(hr_v[r, pl.ds(L, L)], 0.0) * wc[9]
            t += jnp.maximum(tr_v[r, pl.ds(0, L)], 0.0) * wc[10]
            t += jnp.maximum(tr_v[r, pl.ds(L, L)], 0.0) * wc[11]
            acc = jnp.where(lane_iota == rl, _lane_sum(t), acc)
        z = acc + bias
        out_v[pl.ds(row0, L)] = 1.0 / (1.0 + jnp.exp(-z))
        return carry

    lax.fori_loop(0, BPW // L, group_body, 0)
    pltpu.sync_copy(out_v, out_h.at[pl.ds(base, BPW)])


@jax.jit
def _run(x, dr, field, jockey, horse, trainer,
         emb_dr_w, emb_field_w, emb_jockey_w, emb_horse_w, emb_trainer_w,
         W, b):
    w_flat = W.reshape(OUT_DIM).astype(jnp.float32)
    b16 = jnp.broadcast_to(b.reshape(1), (L,)).astype(jnp.float32)
    mesh = plsc.VectorSubcoreMesh(core_axis_name="c", subcore_axis_name="s")
    f = functools.partial(
        pl.kernel, _sc_kernel, mesh=mesh,
        compiler_params=pltpu.CompilerParams(use_tc_tiling_on_sc=False),
        out_type=jax.ShapeDtypeStruct((B,), jnp.float32),
        scratch_types=[
            pltpu.VMEM((BPW, N_NUM_FEATS), jnp.float32),
            pltpu.VMEM((BPW,), jnp.int32),
            pltpu.VMEM((BPW,), jnp.int32),
            pltpu.VMEM((BPW,), jnp.int32),
            pltpu.VMEM((BPW,), jnp.int32),
            pltpu.VMEM((BPW,), jnp.int32),
            pltpu.VMEM((BPW, K_FIELD), jnp.float32),
            pltpu.VMEM((BPW, K_FIELD), jnp.float32),
            pltpu.VMEM((BPW, K_ID), jnp.float32),
            pltpu.VMEM((BPW, K_ID), jnp.float32),
            pltpu.VMEM((BPW, K_ID), jnp.float32),
            pltpu.VMEM((OUT_DIM,), jnp.float32),
            pltpu.VMEM((L,), jnp.float32),
            pltpu.VMEM((BPW,), jnp.float32),
            pltpu.SemaphoreType.DMA,
        ],
    )()
    out = f(x.astype(jnp.float32),
            dr.astype(jnp.int32), field.astype(jnp.int32),
            jockey.astype(jnp.int32), horse.astype(jnp.int32),
            trainer.astype(jnp.int32),
            emb_dr_w, emb_field_w, emb_jockey_w, emb_horse_w, emb_trainer_w,
            w_flat, b16)
    return out.reshape(B, 1)


def kernel(x, dr, field, jockey, horse, trainer, emb_dr_w, emb_field_w,
           emb_jockey_w, emb_horse_w, emb_trainer_w, W, b):
    return _run(x, dr, field, jockey, horse, trainer, emb_dr_w, emb_field_w,
                emb_jockey_w, emb_horse_w, emb_trainer_w, W, b)
